# baseline (speedup 1.0000x reference)
import functools

import jax
import jax.numpy as jnp
from jax import lax
from jax.experimental import pallas as pl
from jax.experimental.pallas import tpu as pltpu
from jax.experimental.pallas import tpu_sc as plsc

VOCAB = 1000000
EMBED = 64
BATCH = 4096
SEQ = 200
NC = 2
NS = 16
NW = NC * NS
RPW = BATCH // NW
RB = 2
LOOKAHEAD = RB - 1
G0 = 128
G1 = SEQ - G0

_mesh = plsc.VectorSubcoreMesh(core_axis_name="c", subcore_axis_name="s")


@functools.partial(
    pl.kernel,
    mesh=_mesh,
    out_type=jax.ShapeDtypeStruct((BATCH, SEQ, EMBED), jnp.float32),
    scratch_types=[
        pltpu.VMEM((RPW, SEQ), jnp.int32),
        pltpu.VMEM((RB, SEQ, EMBED), jnp.float32),
        pltpu.SemaphoreType.DMA((RB,)),
        pltpu.SemaphoreType.DMA((RB,)),
        pltpu.SemaphoreType.DMA((RB,)),
    ],
    compiler_params=pltpu.CompilerParams(
        needs_layout_passes=False, use_tc_tiling_on_sc=False
    ),
)
def _embed(x_hbm, table_hbm, out_hbm, idx_v, rows_v, gsem0, gsem1, wsem):
    wid = lax.axis_index("s") * NC + lax.axis_index("c")
    b0 = wid * RPW

    zeros16 = jnp.zeros((16,), jnp.float32)
    lane = lax.iota(jnp.int32, 16)

    pltpu.sync_copy(x_hbm.at[pl.ds(b0, RPW)], idx_v)

    def fire_gather(i, r):
        pltpu.async_copy(
            table_hbm.at[idx_v.at[i, pl.ds(0, G0)]],
            rows_v.at[r, pl.ds(0, G0)],
            gsem0.at[r],
        )
        pltpu.async_copy(
            table_hbm.at[idx_v.at[i, pl.ds(G0, G1)]],
            rows_v.at[r, pl.ds(G0, G1)],
            gsem1.at[r],
        )

    def wait_gather(i, r):
        pltpu.make_async_copy(
            table_hbm.at[idx_v.at[i, pl.ds(0, G0)]],
            rows_v.at[r, pl.ds(0, G0)],
            gsem0.at[r],
        ).wait()
        pltpu.make_async_copy(
            table_hbm.at[idx_v.at[i, pl.ds(G0, G1)]],
            rows_v.at[r, pl.ds(G0, G1)],
            gsem1.at[r],
        ).wait()

    def fire_write(i, r):
        pltpu.async_copy(rows_v.at[r], out_hbm.at[b0 + i], wsem.at[r])

    def wait_write(i, r):
        pltpu.make_async_copy(
            rows_v.at[r], out_hbm.at[b0 + i], wsem.at[r]
        ).wait()

    def fix(i, r):
        def fix_group(s, fcarry):
            idxv = idx_v[i, pl.ds(s, 16)]
            m = idxv == 0
            nzero = plsc.all_reduce_population_count(m)

            @pl.when(nzero[0] > 0)
            def _zero_rows():
                rows16 = s + lane
                for c in range(EMBED):
                    plsc.store_scatter(
                        rows_v.at[r],
                        [rows16, jnp.full((16,), c, jnp.int32)],
                        zeros16,
                        mask=m,
                    )

            return fcarry

        lax.fori_loop(
            0, SEQ // 16, lambda g, fc: fix_group(g * 16, fc), 0
        )
        fix_group(SEQ - 16, 0)

    for r in range(LOOKAHEAD):
        fire_gather(r, r)

    def body(i, carry):
        r = i % RB
        ia = i + LOOKAHEAD
        ra = ia % RB

        @pl.when(ia < RPW)
        def _ahead():
            @pl.when(ia >= RB)
            def _reuse_wait():
                wait_write(ia - RB, ra)

            fire_gather(ia, ra)

        wait_gather(i, r)
        fix(i, r)
        fire_write(i, r)
        return carry

    lax.fori_loop(0, RPW, body, 0)

    for r in range(RB):
        wait_write(RPW - RB + r, r)


def kernel(x, E):
    return _embed(x.astype(jnp.int32), E)


# speedup vs baseline: 1.0065x; 1.0065x over previous
import functools

import jax
import jax.numpy as jnp
from jax import lax
from jax.experimental import pallas as pl
from jax.experimental.pallas import tpu as pltpu
from jax.experimental.pallas import tpu_sc as plsc

VOCAB = 1000000
EMBED = 64
BATCH = 4096
SEQ = 200
TOTAL = BATCH * SEQ
NC = 2
NS = 16
NW = NC * NS
IPW = TOTAL // NW
G = 128
CHUNK = 512
NG = CHUNK // G
NCH = IPW // CHUNK
RB = 2
LOOKAHEAD = RB - 1

_mesh = plsc.VectorSubcoreMesh(core_axis_name="c", subcore_axis_name="s")


@functools.partial(
    pl.kernel,
    mesh=_mesh,
    out_type=jax.ShapeDtypeStruct((TOTAL, EMBED), jnp.float32),
    scratch_types=[
        pltpu.VMEM((IPW,), jnp.int32),
        pltpu.VMEM((RB, CHUNK, EMBED), jnp.float32),
        pltpu.SemaphoreType.DMA((RB, NG)),
        pltpu.SemaphoreType.DMA((RB,)),
    ],
    compiler_params=pltpu.CompilerParams(
        needs_layout_passes=False, use_tc_tiling_on_sc=False
    ),
)
def _embed(x_hbm, table_hbm, out_hbm, idx_v, rows_v, gsem, wsem):
    wid = lax.axis_index("s") * NC + lax.axis_index("c")
    i0 = wid * IPW

    zeros16 = jnp.zeros((16,), jnp.float32)
    lane = lax.iota(jnp.int32, 16)

    pltpu.sync_copy(x_hbm.at[pl.ds(i0, IPW)], idx_v)

    def fire_gather(i, r):
        for j in range(NG):
            pltpu.async_copy(
                table_hbm.at[idx_v.at[pl.ds(i * CHUNK + j * G, G)]],
                rows_v.at[r, pl.ds(j * G, G)],
                gsem.at[r, j],
            )

    def wait_gather(i, r):
        for j in range(NG):
            pltpu.make_async_copy(
                table_hbm.at[idx_v.at[pl.ds(i * CHUNK + j * G, G)]],
                rows_v.at[r, pl.ds(j * G, G)],
                gsem.at[r, j],
            ).wait()

    def fire_write(i, r):
        pltpu.async_copy(
            rows_v.at[r], out_hbm.at[pl.ds(i0 + i * CHUNK, CHUNK)], wsem.at[r]
        )

    def wait_write(i, r):
        pltpu.make_async_copy(
            rows_v.at[r], out_hbm.at[pl.ds(i0 + i * CHUNK, CHUNK)], wsem.at[r]
        ).wait()

    def fix(i, r):
        def fix_group(g, fcarry):
            idxv = idx_v[pl.ds(i * CHUNK + g * 16, 16)]
            m = idxv == 0
            nzero = plsc.all_reduce_population_count(m)

            @pl.when(nzero[0] > 0)
            def _zero_rows():
                rows16 = g * 16 + lane
                for c in range(EMBED):
                    plsc.store_scatter(
                        rows_v.at[r],
                        [rows16, jnp.full((16,), c, jnp.int32)],
                        zeros16,
                        mask=m,
                    )

            return fcarry

        lax.fori_loop(0, CHUNK // 16, fix_group, 0)

    for r in range(LOOKAHEAD):
        fire_gather(r, r)

    def body(i, carry):
        r = i % RB
        ia = i + LOOKAHEAD
        ra = ia % RB

        @pl.when(ia < NCH)
        def _ahead():
            @pl.when(ia >= RB)
            def _reuse_wait():
                wait_write(ia - RB, ra)

            fire_gather(ia, ra)

        wait_gather(i, r)
        fix(i, r)
        fire_write(i, r)
        return carry

    lax.fori_loop(0, NCH, body, 0)

    for r in range(RB):
        wait_write(NCH - RB + r, r)


def kernel(x, E):
    out = _embed(x.astype(jnp.int32).reshape(TOTAL), E)
    return out.reshape(BATCH, SEQ, EMBED)


# speedup vs baseline: 1.2282x; 1.2203x over previous
import functools

import jax
import jax.numpy as jnp
from jax import lax
from jax.experimental import pallas as pl
from jax.experimental.pallas import tpu as pltpu
from jax.experimental.pallas import tpu_sc as plsc

VOCAB = 1000000
EMBED = 64
EPAD = 128
BATCH = 4096
SEQ = 200
TOTAL = BATCH * SEQ
NC = 2
NS = 16
NW = NC * NS
IPW = TOTAL // NW
G = 128
CHUNK = 256
NG = CHUNK // G
NCH = IPW // CHUNK
RB = 2
LOOKAHEAD = RB - 1

_mesh = plsc.VectorSubcoreMesh(core_axis_name="c", subcore_axis_name="s")


@functools.partial(
    pl.kernel,
    mesh=_mesh,
    out_type=jax.ShapeDtypeStruct((TOTAL, EPAD), jnp.float32),
    scratch_types=[
        pltpu.VMEM((IPW,), jnp.int32),
        pltpu.VMEM((RB, CHUNK, EPAD), jnp.float32),
        pltpu.SemaphoreType.DMA((RB, NG)),
        pltpu.SemaphoreType.DMA((RB,)),
    ],
    compiler_params=pltpu.CompilerParams(
        needs_layout_passes=False, use_tc_tiling_on_sc=False
    ),
)
def _embed(x_hbm, table_hbm, out_hbm, idx_v, rows_v, gsem, wsem):
    wid = lax.axis_index("s") * NC + lax.axis_index("c")
    i0 = wid * IPW

    zeros16 = jnp.zeros((16,), jnp.float32)
    lane = lax.iota(jnp.int32, 16)

    pltpu.sync_copy(x_hbm.at[pl.ds(i0, IPW)], idx_v)

    def fire_gather(i, r):
        for j in range(NG):
            pltpu.async_copy(
                table_hbm.at[idx_v.at[pl.ds(i * CHUNK + j * G, G)]],
                rows_v.at[r, pl.ds(j * G, G)],
                gsem.at[r, j],
            )

    def wait_gather(i, r):
        for j in range(NG):
            pltpu.make_async_copy(
                table_hbm.at[idx_v.at[pl.ds(i * CHUNK + j * G, G)]],
                rows_v.at[r, pl.ds(j * G, G)],
                gsem.at[r, j],
            ).wait()

    def fire_write(i, r):
        pltpu.async_copy(
            rows_v.at[r], out_hbm.at[pl.ds(i0 + i * CHUNK, CHUNK)], wsem.at[r]
        )

    def wait_write(i, r):
        pltpu.make_async_copy(
            rows_v.at[r], out_hbm.at[pl.ds(i0 + i * CHUNK, CHUNK)], wsem.at[r]
        ).wait()

    def fix(i, r):
        def fix_group(g, fcarry):
            idxv = idx_v[pl.ds(i * CHUNK + g * 16, 16)]
            m = idxv == 0
            nzero = plsc.all_reduce_population_count(m)

            @pl.when(nzero[0] > 0)
            def _zero_rows():
                rows16 = g * 16 + lane
                for c in range(EMBED):
                    plsc.store_scatter(
                        rows_v.at[r],
                        [rows16, jnp.full((16,), c, jnp.int32)],
                        zeros16,
                        mask=m,
                    )

            return fcarry

        lax.fori_loop(0, CHUNK // 16, fix_group, 0)

    for r in range(LOOKAHEAD):
        fire_gather(r, r)

    def body(i, carry):
        r = i % RB
        ia = i + LOOKAHEAD
        ra = ia % RB

        @pl.when(ia < NCH)
        def _ahead():
            @pl.when(ia >= RB)
            def _reuse_wait():
                wait_write(ia - RB, ra)

            fire_gather(ia, ra)

        wait_gather(i, r)
        fix(i, r)
        fire_write(i, r)
        return carry

    lax.fori_loop(0, NCH, body, 0)

    for r in range(RB):
        wait_write(NCH - RB + r, r)


def kernel(x, E):
    table = jnp.pad(E, ((0, 0), (0, EPAD - EMBED)))
    out = _embed(x.astype(jnp.int32).reshape(TOTAL), table)
    return out[:, :EMBED].reshape(BATCH, SEQ, EMBED)


# speedup vs baseline: 1.2287x; 1.0004x over previous
import functools

import jax
import jax.numpy as jnp
from jax import lax
from jax.experimental import pallas as pl
from jax.experimental.pallas import tpu as pltpu
from jax.experimental.pallas import tpu_sc as plsc

VOCAB = 1000000
EMBED = 64
EPAD = 128
BATCH = 4096
SEQ = 200
TOTAL = BATCH * SEQ
NC = 2
NS = 16
NW = NC * NS
IPW = TOTAL // NW
G = 128
CHUNK = 256
NG = CHUNK // G
NCH = IPW // CHUNK
RB = 3
LOOKAHEAD = RB - 1

_mesh = plsc.VectorSubcoreMesh(core_axis_name="c", subcore_axis_name="s")


@functools.partial(
    pl.kernel,
    mesh=_mesh,
    out_type=jax.ShapeDtypeStruct((TOTAL, EPAD), jnp.float32),
    scratch_types=[
        pltpu.VMEM((IPW,), jnp.int32),
        pltpu.VMEM((RB, CHUNK, EPAD), jnp.float32),
        pltpu.SemaphoreType.DMA((RB, NG)),
        pltpu.SemaphoreType.DMA((RB,)),
    ],
    compiler_params=pltpu.CompilerParams(
        needs_layout_passes=False, use_tc_tiling_on_sc=False
    ),
)
def _embed(x_hbm, table_hbm, out_hbm, idx_v, rows_v, gsem, wsem):
    wid = lax.axis_index("s") * NC + lax.axis_index("c")
    i0 = wid * IPW

    zeros16 = jnp.zeros((16,), jnp.float32)
    lane = lax.iota(jnp.int32, 16)

    pltpu.sync_copy(x_hbm.at[pl.ds(i0, IPW)], idx_v)

    def fire_gather(i, r):
        for j in range(NG):
            pltpu.async_copy(
                table_hbm.at[idx_v.at[pl.ds(i * CHUNK + j * G, G)]],
                rows_v.at[r, pl.ds(j * G, G)],
                gsem.at[r, j],
            )

    def wait_gather(i, r):
        for j in range(NG):
            pltpu.make_async_copy(
                table_hbm.at[idx_v.at[pl.ds(i * CHUNK + j * G, G)]],
                rows_v.at[r, pl.ds(j * G, G)],
                gsem.at[r, j],
            ).wait()

    def fire_write(i, r):
        pltpu.async_copy(
            rows_v.at[r], out_hbm.at[pl.ds(i0 + i * CHUNK, CHUNK)], wsem.at[r]
        )

    def wait_write(i, r):
        pltpu.make_async_copy(
            rows_v.at[r], out_hbm.at[pl.ds(i0 + i * CHUNK, CHUNK)], wsem.at[r]
        ).wait()

    def fix(i, r):
        def fix_group(g, fcarry):
            idxv = idx_v[pl.ds(i * CHUNK + g * 16, 16)]
            m = idxv == 0
            nzero = plsc.all_reduce_population_count(m)

            @pl.when(nzero[0] > 0)
            def _zero_rows():
                rows16 = g * 16 + lane
                for c in range(EMBED):
                    plsc.store_scatter(
                        rows_v.at[r],
                        [rows16, jnp.full((16,), c, jnp.int32)],
                        zeros16,
                        mask=m,
                    )

            return fcarry

        lax.fori_loop(0, CHUNK // 16, fix_group, 0)

    for r in range(LOOKAHEAD):
        fire_gather(r, r)

    def body(i, carry):
        r = i % RB
        ia = i + LOOKAHEAD
        ra = ia % RB

        @pl.when(ia < NCH)
        def _ahead():
            @pl.when(ia >= RB)
            def _reuse_wait():
                wait_write(ia - RB, ra)

            fire_gather(ia, ra)

        wait_gather(i, r)
        fix(i, r)
        fire_write(i, r)
        return carry

    lax.fori_loop(0, NCH, body, 0)

    for r in range(RB):
        wait_write(NCH - RB + r, r)


def kernel(x, E):
    table = jnp.pad(E, ((0, 0), (0, EPAD - EMBED)))
    out = _embed(x.astype(jnp.int32).reshape(TOTAL), table)
    return out[:, :EMBED].reshape(BATCH, SEQ, EMBED)
